# baseline (speedup 1.0000x reference)
import jax
import jax.numpy as jnp
from jax import lax
from jax.experimental import pallas as pl
from jax.experimental.pallas import tpu as pltpu
from jax.experimental.pallas import tpu_sc as plsc

B = 16384
S = 12
N_ELEMENTS = 118
COMP_SIM_WEIGHT = 2.0

NC = 2
NS = 16
NW = NC * NS
ROWS_PER_W = B // NW
GROUPS = ROWS_PER_W // 16
ACC_WORDS = N_ELEMENTS * 16
ACC_PAD = 1920


def _rsqrt_nr(x):
    i = lax.bitcast_convert_type(x, jnp.int32)
    i = jnp.int32(0x5F3759DF) - lax.shift_right_logical(i, 1)
    y = lax.bitcast_convert_type(i, jnp.float32)
    for _ in range(3):
        y = y * (1.5 - 0.5 * x * y * y)
    return y


def _worker_body(pidx, pfrac, pmask, tidx, tfrac, tmask, out_hbm,
                 pidx_v, pfrac_v, pmask_v, tidx_v, tfrac_v, tmask_v,
                 accp, acct, accp2, acct2, out_v, sem):
    cid = lax.axis_index("c")
    sid = lax.axis_index("s")
    wid = sid * NC + cid
    base = wid * ROWS_PER_W

    copies = [
        pltpu.make_async_copy(hbm.at[:, pl.ds(base, ROWS_PER_W)], v, sem)
        for hbm, v in ((pidx, pidx_v), (pfrac, pfrac_v), (pmask, pmask_v),
                       (tidx, tidx_v), (tfrac, tfrac_v), (tmask, tmask_v))
    ]
    for c in copies:
        c.start()

    zero = jnp.zeros((16,), jnp.float32)

    def clear(i, carry):
        accp[pl.ds(i * 16, 16)] = zero
        acct[pl.ds(i * 16, 16)] = zero
        accp2[pl.ds(i * 16, 16)] = zero
        acct2[pl.ds(i * 16, 16)] = zero
        return carry

    lax.fori_loop(0, ACC_PAD // 16, clear, 0)

    for c in copies:
        c.wait()

    lane_m16 = lax.iota(jnp.int32, 16) - 16

    def one_group(col, cp, ct):
        at, tf = [], []
        for s in range(S):
            at.append(jnp.maximum(tidx_v[s, pl.ds(col, 16)], 1) * 16 + lane_m16)
            tf.append(tfrac_v[s, pl.ds(col, 16)] * tmask_v[s, pl.ds(col, 16)])
        for s in range(S):
            plsc.addupdate_scatter(ct, [at[s]], tf[s])
        tn0 = zero
        tn1 = zero
        tn2_ = zero
        for s in range(0, S, 3):
            tn0 += tf[s] * plsc.load_gather(ct, [at[s]])
            tn1 += tf[s + 1] * plsc.load_gather(ct, [at[s + 1]])
            tn2_ += tf[s + 2] * plsc.load_gather(ct, [at[s + 2]])

        ap, pf = [], []
        for s in range(S):
            ap.append(jnp.maximum(pidx_v[s, pl.ds(col, 16)], 1) * 16 + lane_m16)
            pf.append(pfrac_v[s, pl.ds(col, 16)] * pmask_v[s, pl.ds(col, 16)])
        for s in range(S):
            plsc.addupdate_scatter(cp, [ap[s]], pf[s])
        num0 = zero
        num1 = zero
        pn0 = zero
        pn1 = zero
        for s in range(0, S, 2):
            num0 += pf[s] * plsc.load_gather(ct, [ap[s]])
            num1 += pf[s + 1] * plsc.load_gather(ct, [ap[s + 1]])
            pn0 += pf[s] * plsc.load_gather(cp, [ap[s]])
            pn1 += pf[s + 1] * plsc.load_gather(cp, [ap[s + 1]])

        for s in range(S):
            plsc.store_scatter(cp, [ap[s]], zero)
            plsc.store_scatter(ct, [at[s]], zero)

        num = num0 + num1
        pn2 = pn0 + pn1
        tn2 = (tn0 + tn1) + tn2_
        rp = _rsqrt_nr(jnp.maximum(pn2, 1e-16))
        rt = _rsqrt_nr(jnp.maximum(tn2, 1e-16))
        cos = num * rp * rt
        return cos, pn2 + tn2 - 2.0 * num

    def group(g, carry):
        cos_acc, sse_acc = carry
        col = g * 32
        cos0, sse0 = one_group(col, accp, acct)
        cos1, sse1 = one_group(col + 16, accp2, acct2)
        return cos_acc + (cos0 + cos1), sse_acc + (sse0 + sse1)

    cos_acc, sse_acc = lax.fori_loop(0, GROUPS // 2, group, (zero, zero))
    out_v[0] = cos_acc
    out_v[1] = sse_acc
    pltpu.sync_copy(out_v, out_hbm.at[wid])


def _build(interpret=False):
    mesh = plsc.VectorSubcoreMesh(core_axis_name="c", subcore_axis_name="s",
                                  num_cores=NC, num_subcores=NS)
    return pl.kernel(
        _worker_body,
        out_type=jax.ShapeDtypeStruct((NW, 2, 16), jnp.float32),
        mesh=mesh,
        scratch_types=[
            pltpu.VMEM((S, ROWS_PER_W), jnp.int32),
            pltpu.VMEM((S, ROWS_PER_W), jnp.float32),
            pltpu.VMEM((S, ROWS_PER_W), jnp.float32),
            pltpu.VMEM((S, ROWS_PER_W), jnp.int32),
            pltpu.VMEM((S, ROWS_PER_W), jnp.float32),
            pltpu.VMEM((S, ROWS_PER_W), jnp.float32),
            pltpu.VMEM((ACC_PAD,), jnp.float32),
            pltpu.VMEM((ACC_PAD,), jnp.float32),
            pltpu.VMEM((ACC_PAD,), jnp.float32),
            pltpu.VMEM((ACC_PAD,), jnp.float32),
            pltpu.VMEM((2, 16), jnp.float32),
            pltpu.SemaphoreType.DMA,
        ],
        compiler_params=pltpu.CompilerParams(needs_layout_passes=False),
        interpret=interpret,
        name="composition_vector_loss_sc",
    )


_sc_loss = _build()


@jax.jit
def kernel(pred_element_indices, pred_element_fractions, pred_element_mask,
           target_element_indices, target_element_fractions, target_element_mask):
    partials = _sc_loss(
        pred_element_indices.T,
        pred_element_fractions.T,
        pred_element_mask.astype(jnp.float32).T,
        target_element_indices.T,
        target_element_fractions.T,
        target_element_mask.astype(jnp.float32).T,
    )
    cos_total = jnp.sum(partials[:, 0, :])
    sse_total = jnp.sum(partials[:, 1, :])
    cosine_mean = cos_total / B
    composition_mse = sse_total / (B * N_ELEMENTS)
    composition_loss = (1.0 - cosine_mean) * COMP_SIM_WEIGHT
    return (cosine_mean, composition_mse, composition_loss)


# speedup vs baseline: 1.0423x; 1.0423x over previous
import jax
import jax.numpy as jnp
from jax import lax
from jax.experimental import pallas as pl
from jax.experimental.pallas import tpu as pltpu
from jax.experimental.pallas import tpu_sc as plsc

B = 16384
S = 12
N_ELEMENTS = 118
COMP_SIM_WEIGHT = 2.0

NC = 2
NS = 16
NW = NC * NS
ROWS_PER_W = B // NW
GROUPS = ROWS_PER_W // 16
ACC_WORDS = N_ELEMENTS * 16
ACC_PAD = 1920


def _rsqrt_nr(x):
    i = lax.bitcast_convert_type(x, jnp.int32)
    i = jnp.int32(0x5F3759DF) - lax.shift_right_logical(i, 1)
    y = lax.bitcast_convert_type(i, jnp.float32)
    for _ in range(3):
        y = y * (1.5 - 0.5 * x * y * y)
    return y


def _worker_body(pidx, pfrac, pmask, tidx, tfrac, tmask, out_hbm,
                 pidx_v, pfrac_v, pmask_v, tidx_v, tfrac_v, tmask_v,
                 accp, acct, accp2, acct2, out_v, sem):
    cid = lax.axis_index("c")
    sid = lax.axis_index("s")
    wid = sid * NC + cid
    base = wid * ROWS_PER_W

    copies = [
        pltpu.make_async_copy(hbm.at[:, pl.ds(base, ROWS_PER_W)], v, sem)
        for hbm, v in ((pidx, pidx_v), (pfrac, pfrac_v), (pmask, pmask_v),
                       (tidx, tidx_v), (tfrac, tfrac_v), (tmask, tmask_v))
    ]
    for c in copies:
        c.start()

    zero = jnp.zeros((16,), jnp.float32)

    def clear(i, carry):
        accp[pl.ds(i * 16, 16)] = zero
        acct[pl.ds(i * 16, 16)] = zero
        accp2[pl.ds(i * 16, 16)] = zero
        acct2[pl.ds(i * 16, 16)] = zero
        return carry

    lax.fori_loop(0, ACC_PAD // 16, clear, 0)

    for c in copies:
        c.wait()

    lane_m16 = lax.iota(jnp.int32, 16) - 16

    def one_group(col, cp, ct):
        at, tf = [], []
        for s in range(S):
            at.append(jnp.maximum(tidx_v[s, pl.ds(col, 16)], 1) * 16 + lane_m16)
            tf.append(tfrac_v[s, pl.ds(col, 16)] * tmask_v[s, pl.ds(col, 16)])
        for s in range(S):
            plsc.addupdate_scatter(ct, [at[s]], tf[s])
        tn0 = zero
        tn1 = zero
        tn2_ = zero
        for s in range(0, S, 3):
            tn0 += tf[s] * plsc.load_gather(ct, [at[s]])
            tn1 += tf[s + 1] * plsc.load_gather(ct, [at[s + 1]])
            tn2_ += tf[s + 2] * plsc.load_gather(ct, [at[s + 2]])

        ap, pf = [], []
        for s in range(S):
            ap.append(jnp.maximum(pidx_v[s, pl.ds(col, 16)], 1) * 16 + lane_m16)
            pf.append(pfrac_v[s, pl.ds(col, 16)] * pmask_v[s, pl.ds(col, 16)])
        for s in range(S):
            plsc.addupdate_scatter(cp, [ap[s]], pf[s])
        num0 = zero
        num1 = zero
        pn0 = zero
        pn1 = zero
        for s in range(0, S, 2):
            num0 += pf[s] * plsc.load_gather(ct, [ap[s]])
            num1 += pf[s + 1] * plsc.load_gather(ct, [ap[s + 1]])
            pn0 += pf[s] * plsc.load_gather(cp, [ap[s]])
            pn1 += pf[s + 1] * plsc.load_gather(cp, [ap[s + 1]])

        for s in range(S):
            plsc.store_scatter(cp, [ap[s]], zero)
            plsc.store_scatter(ct, [at[s]], zero)

        num = num0 + num1
        pn2 = pn0 + pn1
        tn2 = (tn0 + tn1) + tn2_
        rp = _rsqrt_nr(jnp.maximum(pn2, 1e-16))
        rt = _rsqrt_nr(jnp.maximum(tn2, 1e-16))
        cos = num * rp * rt
        return cos, pn2 + tn2 - 2.0 * num

    def group(g, carry):
        cos_acc, sse_acc = carry
        cos0, sse0 = one_group(g * 16, accp, acct)
        return cos_acc + cos0, sse_acc + sse0

    cos_acc, sse_acc = lax.fori_loop(0, GROUPS, group, (zero, zero))
    out_v[0] = cos_acc
    out_v[1] = sse_acc
    pltpu.sync_copy(out_v, out_hbm.at[wid])


def _build(interpret=False):
    mesh = plsc.VectorSubcoreMesh(core_axis_name="c", subcore_axis_name="s",
                                  num_cores=NC, num_subcores=NS)
    return pl.kernel(
        _worker_body,
        out_type=jax.ShapeDtypeStruct((NW, 2, 16), jnp.float32),
        mesh=mesh,
        scratch_types=[
            pltpu.VMEM((S, ROWS_PER_W), jnp.int32),
            pltpu.VMEM((S, ROWS_PER_W), jnp.float32),
            pltpu.VMEM((S, ROWS_PER_W), jnp.float32),
            pltpu.VMEM((S, ROWS_PER_W), jnp.int32),
            pltpu.VMEM((S, ROWS_PER_W), jnp.float32),
            pltpu.VMEM((S, ROWS_PER_W), jnp.float32),
            pltpu.VMEM((ACC_PAD,), jnp.float32),
            pltpu.VMEM((ACC_PAD,), jnp.float32),
            pltpu.VMEM((ACC_PAD,), jnp.float32),
            pltpu.VMEM((ACC_PAD,), jnp.float32),
            pltpu.VMEM((2, 16), jnp.float32),
            pltpu.SemaphoreType.DMA,
        ],
        compiler_params=pltpu.CompilerParams(needs_layout_passes=False),
        interpret=interpret,
        name="composition_vector_loss_sc",
    )


_sc_loss = _build()


@jax.jit
def kernel(pred_element_indices, pred_element_fractions, pred_element_mask,
           target_element_indices, target_element_fractions, target_element_mask):
    partials = _sc_loss(
        pred_element_indices.T,
        pred_element_fractions.T,
        pred_element_mask.astype(jnp.float32).T,
        target_element_indices.T,
        target_element_fractions.T,
        target_element_mask.astype(jnp.float32).T,
    )
    cos_total = jnp.sum(partials[:, 0, :])
    sse_total = jnp.sum(partials[:, 1, :])
    cosine_mean = cos_total / B
    composition_mse = sse_total / (B * N_ELEMENTS)
    composition_loss = (1.0 - cosine_mean) * COMP_SIM_WEIGHT
    return (cosine_mean, composition_mse, composition_loss)


# speedup vs baseline: 1.0770x; 1.0333x over previous
import jax
import jax.numpy as jnp
from jax import lax
from jax.experimental import pallas as pl
from jax.experimental.pallas import tpu as pltpu
from jax.experimental.pallas import tpu_sc as plsc

B = 16384
S = 12
N_ELEMENTS = 118
COMP_SIM_WEIGHT = 2.0

NC = 2
NS = 16
NW = NC * NS
ROWS_PER_W = B // NW
GROUPS = ROWS_PER_W // 16
ACC_WORDS = N_ELEMENTS * 16
ACC_PAD = 1920


def _rsqrt_nr(x):
    i = lax.bitcast_convert_type(x, jnp.int32)
    i = jnp.int32(0x5F3759DF) - lax.shift_right_logical(i, 1)
    y = lax.bitcast_convert_type(i, jnp.float32)
    for _ in range(3):
        y = y * (1.5 - 0.5 * x * y * y)
    return y


def _worker_body(pidx, pfrac, tidx, tfrac, out_hbm,
                 pidx_v, pfrac_v, tidx_v, tfrac_v,
                 accp, acct, out_v, sem):
    cid = lax.axis_index("c")
    sid = lax.axis_index("s")
    wid = sid * NC + cid
    base = wid * ROWS_PER_W

    copies = [
        pltpu.make_async_copy(hbm.at[:, pl.ds(base, ROWS_PER_W)], v, sem)
        for hbm, v in ((pidx, pidx_v), (pfrac, pfrac_v),
                       (tidx, tidx_v), (tfrac, tfrac_v))
    ]
    for c in copies:
        c.start()

    zero = jnp.zeros((16,), jnp.float32)

    def clear(i, carry):
        accp[pl.ds(i * 16, 16)] = zero
        acct[pl.ds(i * 16, 16)] = zero
        return carry

    lax.fori_loop(0, ACC_PAD // 16, clear, 0)

    for c in copies:
        c.wait()

    lane_m16 = lax.iota(jnp.int32, 16) - 16

    def one_group(col, cp, ct):
        at, tf = [], []
        for s in range(S):
            at.append(jnp.maximum(tidx_v[s, pl.ds(col, 16)], 1) * 16 + lane_m16)
            tf.append(jnp.maximum(tfrac_v[s, pl.ds(col, 16)], 0.0))
        for s in range(S):
            plsc.addupdate_scatter(ct, [at[s]], tf[s])
        tn0 = zero
        tn1 = zero
        tn2_ = zero
        for s in range(0, S, 3):
            tn0 += tf[s] * plsc.load_gather(ct, [at[s]])
            tn1 += tf[s + 1] * plsc.load_gather(ct, [at[s + 1]])
            tn2_ += tf[s + 2] * plsc.load_gather(ct, [at[s + 2]])

        ap, pf = [], []
        for s in range(S):
            ap.append(jnp.maximum(pidx_v[s, pl.ds(col, 16)], 1) * 16 + lane_m16)
            pf.append(jnp.maximum(pfrac_v[s, pl.ds(col, 16)], 0.0))
        for s in range(S):
            plsc.addupdate_scatter(cp, [ap[s]], pf[s])
        num0 = zero
        num1 = zero
        pn0 = zero
        pn1 = zero
        for s in range(0, S, 2):
            num0 += pf[s] * plsc.load_gather(ct, [ap[s]])
            num1 += pf[s + 1] * plsc.load_gather(ct, [ap[s + 1]])
            pn0 += pf[s] * plsc.load_gather(cp, [ap[s]])
            pn1 += pf[s + 1] * plsc.load_gather(cp, [ap[s + 1]])

        for s in range(S):
            plsc.store_scatter(cp, [ap[s]], zero)
            plsc.store_scatter(ct, [at[s]], zero)

        num = num0 + num1
        pn2 = pn0 + pn1
        tn2 = (tn0 + tn1) + tn2_
        rp = _rsqrt_nr(jnp.maximum(pn2, 1e-16))
        rt = _rsqrt_nr(jnp.maximum(tn2, 1e-16))
        cos = num * rp * rt
        return cos, pn2 + tn2 - 2.0 * num

    def group(g, carry):
        cos_acc, sse_acc = carry
        cos0, sse0 = one_group(g * 16, accp, acct)
        return cos_acc + cos0, sse_acc + sse0

    cos_acc, sse_acc = lax.fori_loop(0, GROUPS, group, (zero, zero))
    out_v[0] = cos_acc
    out_v[1] = sse_acc
    pltpu.sync_copy(out_v, out_hbm.at[wid])


def _build(interpret=False):
    mesh = plsc.VectorSubcoreMesh(core_axis_name="c", subcore_axis_name="s",
                                  num_cores=NC, num_subcores=NS)
    return pl.kernel(
        _worker_body,
        out_type=jax.ShapeDtypeStruct((NW, 2, 16), jnp.float32),
        mesh=mesh,
        scratch_types=[
            pltpu.VMEM((S, ROWS_PER_W), jnp.int32),
            pltpu.VMEM((S, ROWS_PER_W), jnp.float32),
            pltpu.VMEM((S, ROWS_PER_W), jnp.int32),
            pltpu.VMEM((S, ROWS_PER_W), jnp.float32),
            pltpu.VMEM((ACC_PAD,), jnp.float32),
            pltpu.VMEM((ACC_PAD,), jnp.float32),
            pltpu.VMEM((2, 16), jnp.float32),
            pltpu.SemaphoreType.DMA,
        ],
        compiler_params=pltpu.CompilerParams(needs_layout_passes=False),
        interpret=interpret,
        name="composition_vector_loss_sc",
    )


_sc_loss = _build()


@jax.jit
def kernel(pred_element_indices, pred_element_fractions, pred_element_mask,
           target_element_indices, target_element_fractions, target_element_mask):
    pmf = jnp.where(pred_element_mask, pred_element_fractions,
                    -pred_element_fractions)
    tmf = jnp.where(target_element_mask, target_element_fractions,
                    -target_element_fractions)
    partials = _sc_loss(
        pred_element_indices.T,
        pmf.T,
        target_element_indices.T,
        tmf.T,
    )
    cos_total = jnp.sum(partials[:, 0, :])
    sse_total = jnp.sum(partials[:, 1, :])
    cosine_mean = cos_total / B
    composition_mse = sse_total / (B * N_ELEMENTS)
    composition_loss = (1.0 - cosine_mean) * COMP_SIM_WEIGHT
    return (cosine_mean, composition_mse, composition_loss)
